# baseline (speedup 1.0000x reference)
import functools

import jax
import jax.numpy as jnp
from jax import lax
from jax.experimental import pallas as pl
from jax.experimental.pallas import tpu as pltpu
from jax.experimental.pallas import tpu_sc as plsc

N = 10000
D = 128
E = 320000
LAYERS = 3
NC, NS = 2, 16
NW = NC * NS
E_TILE = 10560
BLK = 96
NBLK = E_TILE // BLK
NCH = 11
CB = NBLK // NCH
ACC_ROWS = 10240
ROWS_PER_TILE = ACC_ROWS // NS


def _spmm_body(eds_hbm, vals_hbm, emb_hbm, out_hbm, eds_v, vals_v, rows_v, gsem, ssem, acc_sh):
    c = lax.axis_index("c")
    s = lax.axis_index("s")
    wid = s * NC + c

    zero = jnp.zeros((16,), jnp.float32)

    def zrow(i, carry):
        for q in range(D // 16):
            rows_v[0, i, pl.ds(q * 16, 16)] = zero
        return carry

    lax.fori_loop(0, 64, zrow, 0)
    row0 = s * ROWS_PER_TILE
    for t in range(ROWS_PER_TILE // 64):
        pltpu.async_copy(
            rows_v.at[0, pl.ds(0, 64)],
            acc_sh.at[pl.ds(row0 + t * 64, 64)], ssem)
    for t in range(ROWS_PER_TILE // 64):
        pltpu.make_async_copy(
            rows_v.at[0, pl.ds(0, 64)], acc_sh.at[pl.ds(row0, 64)],
            ssem).wait()
    plsc.subcore_barrier()

    def chunk(ch, carry):
        pltpu.sync_copy(eds_hbm.at[wid, ch], eds_v)
        pltpu.sync_copy(vals_hbm.at[wid, ch], vals_v)
        pltpu.async_copy(emb_hbm.at[eds_v.at[0]], rows_v.at[0], gsem)

        def blk(j, carry2):
            b = j & 1

            @pl.when(j >= 1)
            def _():
                pltpu.make_async_copy(
                    rows_v.at[1 - b], acc_sh.at[eds_v.at[CB + j]], ssem).wait()

            @pl.when(j < CB - 1)
            def _():
                pltpu.async_copy(
                    emb_hbm.at[eds_v.at[j + 1]], rows_v.at[1 - b], gsem)

            pltpu.make_async_copy(
                emb_hbm.at[eds_v.at[j]], rows_v.at[b], gsem).wait()

            for g in range(BLK // 16):
                vv = vals_v[j, pl.ds(g * 16, 16)]
                for l in range(16):
                    e = g * 16 + l
                    vb = vv[l]
                    for q in range(D // 16):
                        rows_v[b, e, pl.ds(q * 16, 16)] = (
                            rows_v[b, e, pl.ds(q * 16, 16)] * vb)
            pltpu.async_copy(
                rows_v.at[b], acc_sh.at[eds_v.at[CB + j]], ssem, add=True)
            return carry2

        lax.fori_loop(0, CB, blk, 0)
        pltpu.make_async_copy(
            rows_v.at[(CB - 1) & 1], acc_sh.at[eds_v.at[2 * CB - 1]], ssem).wait()
        return carry

    lax.fori_loop(0, NCH, chunk, 0)
    plsc.subcore_barrier()

    pltpu.sync_copy(acc_sh.at[pl.ds(row0, ROWS_PER_TILE)],
                    out_hbm.at[c, pl.ds(row0, ROWS_PER_TILE)])


_spmm = functools.partial(
    pl.kernel,
    out_type=jax.ShapeDtypeStruct((NC, ACC_ROWS, D), jnp.float32),
    mesh=plsc.VectorSubcoreMesh(core_axis_name="c", subcore_axis_name="s"),
    scratch_types=[
        pltpu.VMEM((2 * CB, BLK), jnp.int32),
        pltpu.VMEM((CB, BLK), jnp.float32),
        pltpu.VMEM((2, BLK, D), jnp.float32),
        pltpu.SemaphoreType.DMA,
        pltpu.SemaphoreType.DMA,
        pltpu.VMEM_SHARED((ACC_ROWS, D), jnp.float32),
    ],
)(_spmm_body)


def _combine_body(acc_ref, tot_ref, emb_out, tot_out):
    e = acc_ref[0] + acc_ref[1]
    emb_out[...] = e
    tot_out[...] = tot_ref[...] + e


_RB = 1000

_combine = pl.pallas_call(
    _combine_body,
    grid=(N // _RB,),
    in_specs=[
        pl.BlockSpec((NC, _RB, D), lambda i: (0, i, 0)),
        pl.BlockSpec((_RB, D), lambda i: (i, 0)),
    ],
    out_specs=[
        pl.BlockSpec((_RB, D), lambda i: (i, 0)),
        pl.BlockSpec((_RB, D), lambda i: (i, 0)),
    ],
    out_shape=[jax.ShapeDtypeStruct((N, D), jnp.float32)] * 2,
)


def kernel(adj_indices, adj_values, embedding):
    idx = adj_indices.astype(jnp.int32)
    pad = NW * E_TILE - E
    srcf = jnp.pad(idx[1], (0, pad))
    dstf = jnp.pad(idx[0], (0, pad))
    valf = jnp.pad(adj_values, (0, pad))
    eds = jnp.concatenate(
        [srcf.reshape(NW, NCH, CB, BLK),
         dstf.reshape(NW, NCH, CB, BLK)], axis=2)
    vals = valf.reshape(NW, NCH, CB, BLK)
    emb = embedding
    total = embedding
    for _ in range(LAYERS):
        acc = _spmm(eds, vals, emb)
        emb, total = _combine(acc, total)
    return total


# speedup vs baseline: 4.4067x; 4.4067x over previous
import functools

import jax
import jax.numpy as jnp
from jax import lax
from jax.experimental import pallas as pl
from jax.experimental.pallas import tpu as pltpu
from jax.experimental.pallas import tpu_sc as plsc

N = 10000
D = 128
E = 320000
LAYERS = 3
NC, NS = 2, 16
NW = NC * NS
E_TILE = 10560
BLK = 96
NBLK = E_TILE // BLK
NCH = 11
CB = NBLK // NCH
ACC_ROWS = 10240
ROWS_PER_TILE = ACC_ROWS // NS


def _spmm_body(eds_hbm, vals_hbm, emb_hbm, out_hbm, eds_v, vals_v, rows_v, gsem, ssem, acc_sh):
    c = lax.axis_index("c")
    s = lax.axis_index("s")
    wid = s * NC + c

    zero = jnp.zeros((16,), jnp.float32)

    def zrow(i, carry):
        for q in range(D // 16):
            rows_v[0, i, pl.ds(q * 16, 16)] = zero
        return carry

    lax.fori_loop(0, 64, zrow, 0)
    row0 = s * ROWS_PER_TILE
    for t in range(ROWS_PER_TILE // 64):
        pltpu.async_copy(
            rows_v.at[0, pl.ds(0, 64)],
            acc_sh.at[pl.ds(row0 + t * 64, 64)], ssem)
    for t in range(ROWS_PER_TILE // 64):
        pltpu.make_async_copy(
            rows_v.at[0, pl.ds(0, 64)], acc_sh.at[pl.ds(row0, 64)],
            ssem).wait()
    plsc.subcore_barrier()

    def chunk(ch, carry):
        pltpu.sync_copy(eds_hbm.at[wid, ch], eds_v)
        pltpu.sync_copy(vals_hbm.at[wid, ch], vals_v)
        pltpu.async_copy(emb_hbm.at[eds_v.at[0]], rows_v.at[0], gsem)

        def blk(j, carry2):
            b = j & 1

            @pl.when(j >= 1)
            def _():
                pltpu.make_async_copy(
                    rows_v.at[1 - b], acc_sh.at[eds_v.at[CB + j]], ssem).wait()

            @pl.when(j < CB - 1)
            def _():
                pltpu.async_copy(
                    emb_hbm.at[eds_v.at[j + 1]], rows_v.at[1 - b], gsem)

            pltpu.make_async_copy(
                emb_hbm.at[eds_v.at[j]], rows_v.at[b], gsem).wait()

            for g in range(BLK // 16):
                vv = vals_v[j, pl.ds(g * 16, 16)]
                for l in range(16):
                    e = g * 16 + l
                    vb = vv[l]
                    for q in range(D // 16):
                        rows_v[b, e, pl.ds(q * 16, 16)] = (
                            rows_v[b, e, pl.ds(q * 16, 16)] * vb)
            pltpu.async_copy(
                rows_v.at[b], acc_sh.at[eds_v.at[CB + j]], ssem, add=True)
            return carry2

        lax.fori_loop(0, CB, blk, 0)
        pltpu.make_async_copy(
            rows_v.at[(CB - 1) & 1], acc_sh.at[eds_v.at[2 * CB - 1]], ssem).wait()
        return carry

    lax.fori_loop(0, NCH, chunk, 0)
    plsc.subcore_barrier()

    pltpu.sync_copy(acc_sh.at[pl.ds(row0, ROWS_PER_TILE)],
                    out_hbm.at[c, pl.ds(row0, ROWS_PER_TILE)])


_spmm = functools.partial(
    pl.kernel,
    out_type=jax.ShapeDtypeStruct((NC, ACC_ROWS, D), jnp.float32),
    mesh=plsc.VectorSubcoreMesh(core_axis_name="c", subcore_axis_name="s"),
    scratch_types=[
        pltpu.VMEM((2 * CB, BLK), jnp.int32),
        pltpu.VMEM((CB, BLK), jnp.float32),
        pltpu.VMEM((2, BLK, D), jnp.float32),
        pltpu.SemaphoreType.DMA,
        pltpu.SemaphoreType.DMA,
        pltpu.VMEM_SHARED((ACC_ROWS, D), jnp.float32),
    ],
)(_spmm_body)


def _combine_body(acc_ref, tot_ref, emb_out, tot_out):
    e = acc_ref[0] + acc_ref[1]
    emb_out[...] = e
    tot_out[...] = tot_ref[...] + e


_RB = 1000

_combine = pl.pallas_call(
    _combine_body,
    grid=(N // _RB,),
    in_specs=[
        pl.BlockSpec((NC, _RB, D), lambda i: (0, i, 0)),
        pl.BlockSpec((_RB, D), lambda i: (i, 0)),
    ],
    out_specs=[
        pl.BlockSpec((_RB, D), lambda i: (i, 0)),
        pl.BlockSpec((_RB, D), lambda i: (i, 0)),
    ],
    out_shape=[jax.ShapeDtypeStruct((N, D), jnp.float32)] * 2,
)


def kernel(adj_indices, adj_values, embedding):
    idx = adj_indices.astype(jnp.int32)
    pad = NW * E_TILE - E
    spread = (jnp.arange(pad, dtype=jnp.int32) * 37) % N
    srcf = jnp.concatenate([idx[1], spread])
    dstf = jnp.concatenate([idx[0], spread])
    valf = jnp.pad(adj_values, (0, pad))
    eds = jnp.concatenate(
        [srcf.reshape(NW, NCH, CB, BLK),
         dstf.reshape(NW, NCH, CB, BLK)], axis=2)
    vals = valf.reshape(NW, NCH, CB, BLK)
    emb = embedding
    total = embedding
    for _ in range(LAYERS):
        acc = _spmm(eds, vals, emb)
        emb, total = _combine(acc, total)
    return total


# speedup vs baseline: 4.8892x; 1.1095x over previous
import functools

import jax
import jax.numpy as jnp
from jax import lax
from jax.experimental import pallas as pl
from jax.experimental.pallas import tpu as pltpu
from jax.experimental.pallas import tpu_sc as plsc

N = 10000
D = 128
E = 320000
LAYERS = 3
NC, NS = 2, 16
NW = NC * NS
E_TILE = 10240
BLK = 80
NBLK = E_TILE // BLK
NCH = 8
CB = NBLK // NCH
ACC_ROWS = 10240
ROWS_PER_TILE = ACC_ROWS // NS


def _spmm_body(eds_hbm, vals_hbm, emb_hbm, out_hbm,
               eds_v, vals_v, rows_v, gsem, ssem, stsem, acc_sh):
    c = lax.axis_index("c")
    s = lax.axis_index("s")
    wid = s * NC + c

    zero = jnp.zeros((16,), jnp.float32)

    def zrow(i, carry):
        for q in range(D // 16):
            rows_v[0, i, pl.ds(q * 16, 16)] = zero
        return carry

    lax.fori_loop(0, BLK, zrow, 0)
    row0 = s * ROWS_PER_TILE
    for t in range(ROWS_PER_TILE // BLK):
        pltpu.async_copy(
            rows_v.at[0], acc_sh.at[pl.ds(row0 + t * BLK, BLK)], ssem)
    for t in range(ROWS_PER_TILE // BLK):
        pltpu.make_async_copy(
            rows_v.at[0], acc_sh.at[pl.ds(row0, BLK)], ssem).wait()
    plsc.subcore_barrier()

    pltpu.sync_copy(eds_hbm.at[wid, 0], eds_v.at[pl.ds(0, 2 * CB)])
    pltpu.sync_copy(vals_hbm.at[wid, 0], vals_v.at[pl.ds(0, CB)])
    pltpu.async_copy(eds_hbm.at[wid, 1], eds_v.at[pl.ds(2 * CB, 2 * CB)],
                     stsem)
    pltpu.async_copy(vals_hbm.at[wid, 1], vals_v.at[pl.ds(CB, CB)], stsem)
    pltpu.async_copy(emb_hbm.at[eds_v.at[0]], rows_v.at[0], gsem)

    def chunk(ch, carry):
        p = lax.rem(ch, 2)
        base = p * 2 * CB
        vbase = p * CB

        def blk(j, carry2):
            b = j & 1

            @pl.when(j >= 1)
            def _():
                pltpu.make_async_copy(
                    rows_v.at[1 - b], acc_sh.at[eds_v.at[base + CB + j]],
                    ssem).wait()

            @pl.when(j < CB - 1)
            def _():
                pltpu.async_copy(
                    emb_hbm.at[eds_v.at[base + j + 1]], rows_v.at[1 - b],
                    gsem)

            @pl.when(jnp.logical_and(j == CB - 1, ch < NCH - 1))
            def _():
                pltpu.make_async_copy(
                    eds_hbm.at[wid, 0],
                    eds_v.at[pl.ds((1 - p) * 2 * CB, 2 * CB)], stsem).wait()
                pltpu.make_async_copy(
                    vals_hbm.at[wid, 0],
                    vals_v.at[pl.ds((1 - p) * CB, CB)], stsem).wait()
                pltpu.async_copy(
                    emb_hbm.at[eds_v.at[(1 - p) * 2 * CB]], rows_v.at[1 - b],
                    gsem)

            pltpu.make_async_copy(
                emb_hbm.at[eds_v.at[base + j]], rows_v.at[b], gsem).wait()

            for g in range(BLK // 16):
                vv = vals_v[vbase + j, pl.ds(g * 16, 16)]
                for l in range(16):
                    e = g * 16 + l
                    vb = vv[l]
                    for q in range(D // 16):
                        rows_v[b, e, pl.ds(q * 16, 16)] = (
                            rows_v[b, e, pl.ds(q * 16, 16)] * vb)
            pltpu.async_copy(
                rows_v.at[b], acc_sh.at[eds_v.at[base + CB + j]], ssem,
                add=True)
            return carry2

        lax.fori_loop(0, CB, blk, 0)
        pltpu.make_async_copy(
            rows_v.at[(CB - 1) & 1], acc_sh.at[eds_v.at[base + 2 * CB - 1]],
            ssem).wait()

        @pl.when(ch + 2 < NCH)
        def _():
            pltpu.async_copy(
                eds_hbm.at[wid, ch + 2], eds_v.at[pl.ds(base, 2 * CB)], stsem)
            pltpu.async_copy(
                vals_hbm.at[wid, ch + 2], vals_v.at[pl.ds(vbase, CB)], stsem)

        return carry

    lax.fori_loop(0, NCH, chunk, 0)
    plsc.subcore_barrier()

    pltpu.sync_copy(acc_sh.at[pl.ds(row0, ROWS_PER_TILE)],
                    out_hbm.at[c, pl.ds(row0, ROWS_PER_TILE)])


_spmm = functools.partial(
    pl.kernel,
    out_type=jax.ShapeDtypeStruct((NC, ACC_ROWS, D), jnp.float32),
    mesh=plsc.VectorSubcoreMesh(core_axis_name="c", subcore_axis_name="s"),
    scratch_types=[
        pltpu.VMEM((4 * CB, BLK), jnp.int32),
        pltpu.VMEM((2 * CB, BLK), jnp.float32),
        pltpu.VMEM((2, BLK, D), jnp.float32),
        pltpu.SemaphoreType.DMA,
        pltpu.SemaphoreType.DMA,
        pltpu.SemaphoreType.DMA,
        pltpu.VMEM_SHARED((ACC_ROWS, D), jnp.float32),
    ],
)(_spmm_body)


def _combine_body(acc_ref, tot_ref, emb_out, tot_out):
    e = acc_ref[0] + acc_ref[1]
    emb_out[...] = e
    tot_out[...] = tot_ref[...] + e


_RB = 1000

_combine = pl.pallas_call(
    _combine_body,
    grid=(N // _RB,),
    in_specs=[
        pl.BlockSpec((NC, _RB, D), lambda i: (0, i, 0)),
        pl.BlockSpec((_RB, D), lambda i: (i, 0)),
    ],
    out_specs=[
        pl.BlockSpec((_RB, D), lambda i: (i, 0)),
        pl.BlockSpec((_RB, D), lambda i: (i, 0)),
    ],
    out_shape=[jax.ShapeDtypeStruct((N, D), jnp.float32)] * 2,
)


def kernel(adj_indices, adj_values, embedding):
    idx = adj_indices.astype(jnp.int32)
    pad = NW * E_TILE - E
    spread = (jnp.arange(pad, dtype=jnp.int32) * 37) % N
    srcf = jnp.concatenate([idx[1], spread])
    dstf = jnp.concatenate([idx[0], spread])
    valf = jnp.pad(adj_values, (0, pad))
    eds = jnp.concatenate(
        [srcf.reshape(NW, NCH, CB, BLK),
         dstf.reshape(NW, NCH, CB, BLK)], axis=2)
    vals = valf.reshape(NW, NCH, CB, BLK)
    emb = embedding
    total = embedding
    for _ in range(LAYERS):
        acc = _spmm(eds, vals, emb)
        emb, total = _combine(acc, total)
    return total


# speedup vs baseline: 4.9385x; 1.0101x over previous
import functools

import jax
import jax.numpy as jnp
from jax import lax
from jax.experimental import pallas as pl
from jax.experimental.pallas import tpu as pltpu
from jax.experimental.pallas import tpu_sc as plsc

N = 10000
D = 128
E = 320000
LAYERS = 3
NC, NS = 2, 16
NW = NC * NS
E_TILE = 10240
BLK = 80
NBLK = E_TILE // BLK
NCH = 8
CB = NBLK // NCH
ACC_ROWS = 10240
ROWS_PER_TILE = ACC_ROWS // NS


def _spmm_body(eds_hbm, vals_hbm, emb_hbm, out_hbm,
               eds_v, vals_v, rows_v, gsem, ssem, stsem, acc_sh):
    c = lax.axis_index("c")
    s = lax.axis_index("s")
    wid = s * NC + c

    zero = jnp.zeros((16,), jnp.float32)

    def zrow(i, carry):
        for q in range(D // 16):
            rows_v[0, i, pl.ds(q * 16, 16)] = zero
        return carry

    lax.fori_loop(0, BLK, zrow, 0)
    row0 = s * ROWS_PER_TILE
    pltpu.async_copy(eds_hbm.at[wid, 0], eds_v.at[pl.ds(0, 2 * CB)], stsem)
    pltpu.async_copy(vals_hbm.at[wid, 0], vals_v.at[pl.ds(0, CB)], stsem)
    pltpu.async_copy(eds_hbm.at[wid, 1], eds_v.at[pl.ds(2 * CB, 2 * CB)],
                     stsem)
    pltpu.async_copy(vals_hbm.at[wid, 1], vals_v.at[pl.ds(CB, CB)], stsem)
    for t in range(ROWS_PER_TILE // BLK):
        pltpu.async_copy(
            rows_v.at[0], acc_sh.at[pl.ds(row0 + t * BLK, BLK)], ssem)
    for t in range(ROWS_PER_TILE // BLK):
        pltpu.make_async_copy(
            rows_v.at[0], acc_sh.at[pl.ds(row0, BLK)], ssem).wait()
    pltpu.make_async_copy(
        eds_hbm.at[wid, 0], eds_v.at[pl.ds(0, 2 * CB)], stsem).wait()
    pltpu.make_async_copy(
        vals_hbm.at[wid, 0], vals_v.at[pl.ds(0, CB)], stsem).wait()
    plsc.subcore_barrier()
    pltpu.async_copy(emb_hbm.at[eds_v.at[0]], rows_v.at[0], gsem)

    def chunk(ch, carry):
        p = lax.rem(ch, 2)
        base = p * 2 * CB
        vbase = p * CB

        def blk(j, carry2):
            b = j & 1

            @pl.when(j >= 1)
            def _():
                pltpu.make_async_copy(
                    rows_v.at[1 - b], acc_sh.at[eds_v.at[base + CB + j]],
                    ssem).wait()

            @pl.when(j < CB - 1)
            def _():
                pltpu.async_copy(
                    emb_hbm.at[eds_v.at[base + j + 1]], rows_v.at[1 - b],
                    gsem)

            @pl.when(jnp.logical_and(j == CB - 1, ch < NCH - 1))
            def _():
                pltpu.make_async_copy(
                    eds_hbm.at[wid, 0],
                    eds_v.at[pl.ds((1 - p) * 2 * CB, 2 * CB)], stsem).wait()
                pltpu.make_async_copy(
                    vals_hbm.at[wid, 0],
                    vals_v.at[pl.ds((1 - p) * CB, CB)], stsem).wait()
                pltpu.async_copy(
                    emb_hbm.at[eds_v.at[(1 - p) * 2 * CB]], rows_v.at[1 - b],
                    gsem)

            pltpu.make_async_copy(
                emb_hbm.at[eds_v.at[base + j]], rows_v.at[b], gsem).wait()

            for g in range(BLK // 16):
                vv = vals_v[vbase + j, pl.ds(g * 16, 16)]
                for l in range(16):
                    e = g * 16 + l
                    vb = vv[l]
                    for q in range(D // 16):
                        rows_v[b, e, pl.ds(q * 16, 16)] = (
                            rows_v[b, e, pl.ds(q * 16, 16)] * vb)
            pltpu.async_copy(
                rows_v.at[b], acc_sh.at[eds_v.at[base + CB + j]], ssem,
                add=True)
            return carry2

        lax.fori_loop(0, CB, blk, 0)
        pltpu.make_async_copy(
            rows_v.at[(CB - 1) & 1], acc_sh.at[eds_v.at[base + 2 * CB - 1]],
            ssem).wait()

        @pl.when(ch + 2 < NCH)
        def _():
            pltpu.async_copy(
                eds_hbm.at[wid, ch + 2], eds_v.at[pl.ds(base, 2 * CB)], stsem)
            pltpu.async_copy(
                vals_hbm.at[wid, ch + 2], vals_v.at[pl.ds(vbase, CB)], stsem)

        return carry

    lax.fori_loop(0, NCH, chunk, 0)
    plsc.subcore_barrier()

    pltpu.sync_copy(acc_sh.at[pl.ds(row0, ROWS_PER_TILE)],
                    out_hbm.at[c, pl.ds(row0, ROWS_PER_TILE)])


_spmm = functools.partial(
    pl.kernel,
    out_type=jax.ShapeDtypeStruct((NC, ACC_ROWS, D), jnp.float32),
    mesh=plsc.VectorSubcoreMesh(core_axis_name="c", subcore_axis_name="s"),
    scratch_types=[
        pltpu.VMEM((4 * CB, BLK), jnp.int32),
        pltpu.VMEM((2 * CB, BLK), jnp.float32),
        pltpu.VMEM((2, BLK, D), jnp.float32),
        pltpu.SemaphoreType.DMA,
        pltpu.SemaphoreType.DMA,
        pltpu.SemaphoreType.DMA,
        pltpu.VMEM_SHARED((ACC_ROWS, D), jnp.float32),
    ],
)(_spmm_body)


def _combine_body(acc_ref, tot_ref, emb_out, tot_out):
    e = acc_ref[0] + acc_ref[1]
    emb_out[...] = e
    tot_out[...] = tot_ref[...] + e


_RB = 1000

_combine = pl.pallas_call(
    _combine_body,
    grid=(N // _RB,),
    in_specs=[
        pl.BlockSpec((NC, _RB, D), lambda i: (0, i, 0)),
        pl.BlockSpec((_RB, D), lambda i: (i, 0)),
    ],
    out_specs=[
        pl.BlockSpec((_RB, D), lambda i: (i, 0)),
        pl.BlockSpec((_RB, D), lambda i: (i, 0)),
    ],
    out_shape=[jax.ShapeDtypeStruct((N, D), jnp.float32)] * 2,
)


def kernel(adj_indices, adj_values, embedding):
    idx = adj_indices.astype(jnp.int32)
    pad = NW * E_TILE - E
    spread = (jnp.arange(pad, dtype=jnp.int32) * 37) % N
    srcf = jnp.concatenate([idx[1], spread])
    dstf = jnp.concatenate([idx[0], spread])
    valf = jnp.pad(adj_values, (0, pad))
    eds = jnp.concatenate(
        [srcf.reshape(NW, NCH, CB, BLK),
         dstf.reshape(NW, NCH, CB, BLK)], axis=2)
    vals = valf.reshape(NW, NCH, CB, BLK)
    emb = embedding
    total = embedding
    for _ in range(LAYERS):
        acc = _spmm(eds, vals, emb)
        emb, total = _combine(acc, total)
    return total


# speedup vs baseline: 4.9391x; 1.0001x over previous
import functools

import jax
import jax.numpy as jnp
from jax import lax
from jax.experimental import pallas as pl
from jax.experimental.pallas import tpu as pltpu
from jax.experimental.pallas import tpu_sc as plsc

N = 10000
D = 128
E = 320000
LAYERS = 3
NC, NS = 2, 16
NW = NC * NS
E_TILE = 10240
BLK = 80
NBLK = E_TILE // BLK
NCH = 8
CB = NBLK // NCH
ACC_ROWS = 10240
ROWS_PER_TILE = ACC_ROWS // NS


def _spmm_body(eds_hbm, vals_hbm, emb_hbm, out_hbm,
               eds_v, vals_v, rows_v, gsem, ssem, stsem, acc_sh):
    c = lax.axis_index("c")
    s = lax.axis_index("s")
    wid = s * NC + c

    zero = jnp.zeros((16,), jnp.float32)

    def zrow(i, carry):
        for q in range(D // 16):
            rows_v[0, i, pl.ds(q * 16, 16)] = zero
        return carry

    lax.fori_loop(0, BLK, zrow, 0)
    row0 = s * ROWS_PER_TILE
    pltpu.async_copy(eds_hbm.at[wid, 0], eds_v.at[pl.ds(0, 2 * CB)], stsem)
    pltpu.async_copy(vals_hbm.at[wid, 0], vals_v.at[pl.ds(0, CB)], stsem)
    pltpu.async_copy(eds_hbm.at[wid, 1], eds_v.at[pl.ds(2 * CB, 2 * CB)],
                     stsem)
    pltpu.async_copy(vals_hbm.at[wid, 1], vals_v.at[pl.ds(CB, CB)], stsem)
    for t in range(ROWS_PER_TILE // BLK):
        pltpu.async_copy(
            rows_v.at[0], acc_sh.at[pl.ds(row0 + t * BLK, BLK)], ssem)
    for t in range(ROWS_PER_TILE // BLK):
        pltpu.make_async_copy(
            rows_v.at[0], acc_sh.at[pl.ds(row0, BLK)], ssem).wait()
    pltpu.make_async_copy(
        eds_hbm.at[wid, 0], eds_v.at[pl.ds(0, 2 * CB)], stsem).wait()
    pltpu.make_async_copy(
        vals_hbm.at[wid, 0], vals_v.at[pl.ds(0, CB)], stsem).wait()
    plsc.subcore_barrier()
    pltpu.async_copy(emb_hbm.at[eds_v.at[0]], rows_v.at[0], gsem)

    def chunk(ch, carry):
        p = lax.rem(ch, 2)
        base = p * 2 * CB
        vbase = p * CB

        def blk(j, carry2):
            b = j & 1

            @pl.when(j >= 1)
            def _():
                pltpu.make_async_copy(
                    rows_v.at[1 - b], acc_sh.at[eds_v.at[base + CB + j]],
                    ssem).wait()

            @pl.when(j < CB - 1)
            def _():
                pltpu.async_copy(
                    emb_hbm.at[eds_v.at[base + j + 1]], rows_v.at[1 - b],
                    gsem)

            @pl.when(jnp.logical_and(j == CB - 1, ch < NCH - 1))
            def _():
                pltpu.make_async_copy(
                    eds_hbm.at[wid, 0],
                    eds_v.at[pl.ds((1 - p) * 2 * CB, 2 * CB)], stsem).wait()
                pltpu.make_async_copy(
                    vals_hbm.at[wid, 0],
                    vals_v.at[pl.ds((1 - p) * CB, CB)], stsem).wait()
                pltpu.async_copy(
                    emb_hbm.at[eds_v.at[(1 - p) * 2 * CB]], rows_v.at[1 - b],
                    gsem)

            pltpu.make_async_copy(
                emb_hbm.at[eds_v.at[base + j]], rows_v.at[b], gsem).wait()

            for g in range(BLK // 16):
                vv = vals_v[vbase + j, pl.ds(g * 16, 16)]
                for l in range(16):
                    e = g * 16 + l
                    vb = vv[l]
                    for q in range(D // 16):
                        rows_v[b, e, pl.ds(q * 16, 16)] = (
                            rows_v[b, e, pl.ds(q * 16, 16)] * vb)
            pltpu.async_copy(
                rows_v.at[b], acc_sh.at[eds_v.at[base + CB + j]], ssem,
                add=True)
            return carry2

        lax.fori_loop(0, CB, blk, 0)
        pltpu.make_async_copy(
            rows_v.at[(CB - 1) & 1], acc_sh.at[eds_v.at[base + 2 * CB - 1]],
            ssem).wait()

        @pl.when(ch + 2 < NCH)
        def _():
            pltpu.async_copy(
                eds_hbm.at[wid, ch + 2], eds_v.at[pl.ds(base, 2 * CB)], stsem)
            pltpu.async_copy(
                vals_hbm.at[wid, ch + 2], vals_v.at[pl.ds(vbase, CB)], stsem)

        return carry

    lax.fori_loop(0, NCH, chunk, 0)
    plsc.subcore_barrier()

    pltpu.sync_copy(acc_sh.at[pl.ds(row0, ROWS_PER_TILE)],
                    out_hbm.at[c, pl.ds(row0, ROWS_PER_TILE)])


_spmm = functools.partial(
    pl.kernel,
    out_type=jax.ShapeDtypeStruct((NC, ACC_ROWS, D), jnp.float32),
    mesh=plsc.VectorSubcoreMesh(core_axis_name="c", subcore_axis_name="s"),
    scratch_types=[
        pltpu.VMEM((4 * CB, BLK), jnp.int32),
        pltpu.VMEM((2 * CB, BLK), jnp.float32),
        pltpu.VMEM((2, BLK, D), jnp.float32),
        pltpu.SemaphoreType.DMA,
        pltpu.SemaphoreType.DMA,
        pltpu.SemaphoreType.DMA,
        pltpu.VMEM_SHARED((ACC_ROWS, D), jnp.float32),
    ],
)(_spmm_body)


def _combine_body(acc_ref, tot_ref, emb_out, tot_out):
    e = acc_ref[0] + acc_ref[1]
    emb_out[...] = e
    tot_out[...] = tot_ref[...] + e


_RB = 1000

_combine = pl.pallas_call(
    _combine_body,
    grid=(N // _RB,),
    in_specs=[
        pl.BlockSpec((NC, _RB, D), lambda i: (0, i, 0)),
        pl.BlockSpec((_RB, D), lambda i: (i, 0)),
    ],
    out_specs=[
        pl.BlockSpec((_RB, D), lambda i: (i, 0)),
        pl.BlockSpec((_RB, D), lambda i: (i, 0)),
    ],
    out_shape=[jax.ShapeDtypeStruct((N, D), jnp.float32)] * 2,
)


def _combine_last_body(acc_ref, tot_ref, tot_out):
    tot_out[...] = tot_ref[...] + acc_ref[0] + acc_ref[1]


_combine_last = pl.pallas_call(
    _combine_last_body,
    grid=(N // _RB,),
    in_specs=[
        pl.BlockSpec((NC, _RB, D), lambda i: (0, i, 0)),
        pl.BlockSpec((_RB, D), lambda i: (i, 0)),
    ],
    out_specs=pl.BlockSpec((_RB, D), lambda i: (i, 0)),
    out_shape=jax.ShapeDtypeStruct((N, D), jnp.float32),
)


def kernel(adj_indices, adj_values, embedding):
    idx = adj_indices.astype(jnp.int32)
    pad = NW * E_TILE - E
    spread = (jnp.arange(pad, dtype=jnp.int32) * 37) % N
    srcf = jnp.concatenate([idx[1], spread])
    dstf = jnp.concatenate([idx[0], spread])
    valf = jnp.pad(adj_values, (0, pad))
    eds = jnp.concatenate(
        [srcf.reshape(NW, NCH, CB, BLK),
         dstf.reshape(NW, NCH, CB, BLK)], axis=2)
    vals = valf.reshape(NW, NCH, CB, BLK)
    emb = embedding
    total = embedding
    for layer in range(LAYERS):
        acc = _spmm(eds, vals, emb)
        if layer < LAYERS - 1:
            emb, total = _combine(acc, total)
        else:
            total = _combine_last(acc, total)
    return total
